# Initial kernel scaffold; baseline (speedup 1.0000x reference)
import jax
import jax.numpy as jnp
from jax.experimental import pallas as pl


def kernel(name, collaborative, track_uri_can, n_songs_pl, num_artists_pl, num_albums_pl, artist_name_pl, track_uri_pl, track_name_pl, duration_ms_songs_pl, album_name_pl, artist_pop_pl, artists_followers_pl, track_pop_pl, artist_genres_pl, params):
    raise NotImplementedError("write your pallas kernel here")


# speedup vs baseline: 18.3534x; 18.3534x over previous
import functools

import numpy as np
import jax
import jax.numpy as jnp
from jax import lax
from jax.experimental import pallas as pl
from jax.experimental.pallas import tpu as pltpu
from jax.experimental.pallas import tpu_sc as plsc

B = 4096
L = 20
EMB = 128
NW = 32
BPW = B // NW
CHUNK = 128
NCHUNK = (BPW * L) // CHUNK
NPOOL = 5

NB = np.linspace(1.0, 5.0, 5).astype(np.float32)
DUR_B = np.linspace(-1.0, 20744575.0, 100).astype(np.float32)
FOL_B = np.linspace(0.0, 94437255.0, 10).astype(np.float32)
TPOP_B = np.linspace(-1.0, 96.0, 10).astype(np.float32)


def _sc_body(idx_hbm, dest_hbm, zeros_hbm,
             ta, tu, tt, tal, tg, tn, tc,
             o_a, o_u, o_t, o_al, o_g, o_n, o_c,
             idx_v, dest_v, acc, gbuf, sem):
    c = lax.axis_index("c")
    s = lax.axis_index("s")
    w = s * 2 + c
    sbase = s * (NPOOL * BPW)
    pltpu.sync_copy(idx_hbm.at[w], idx_v)
    pltpu.sync_copy(dest_hbm.at[s], dest_v)
    pltpu.sync_copy(zeros_hbm, acc.at[pl.ds(sbase, NPOOL * BPW)])

    tables = [ta, tu, tt, tal, tg]
    for f in range(NPOOL):
        tbl = tables[f]

        def chunk_body(j, _, tbl=tbl, f=f):
            r = f * NCHUNK + j
            pltpu.async_copy(tbl.at[idx_v.at[r]], gbuf, sem).wait()
            pltpu.sync_copy(gbuf, acc.at[dest_v.at[r]], add=True)
            return 0

        lax.fori_loop(0, NCHUNK, chunk_body, 0)

    outs = [o_a, o_u, o_t, o_al, o_g]
    row0 = w * BPW
    for f in range(NPOOL):
        pltpu.sync_copy(acc.at[pl.ds(sbase + f * BPW, BPW)],
                        outs[f].at[pl.ds(row0, BPW)])

    pltpu.async_copy(tn.at[idx_v.at[NPOOL * NCHUNK]], gbuf, sem).wait()
    pltpu.sync_copy(gbuf, o_n.at[pl.ds(row0, BPW)])
    pltpu.async_copy(tc.at[idx_v.at[NPOOL * NCHUNK + 1]], gbuf, sem).wait()
    pltpu.sync_copy(gbuf, o_c.at[pl.ds(row0, BPW)])


def _sc_gather(idx_all, dest, zeros, ta, tu, tt, tal, tg, tn, tc):
    out = tuple(jax.ShapeDtypeStruct((B, EMB), jnp.float32) for _ in range(7))
    mesh = plsc.VectorSubcoreMesh(core_axis_name="c", subcore_axis_name="s")
    f = pl.kernel(
        _sc_body,
        out_type=out,
        mesh=mesh,
        scratch_types=[
            pltpu.VMEM((NPOOL * NCHUNK + 2, CHUNK), jnp.int32),
            pltpu.VMEM((NPOOL * NCHUNK, CHUNK), jnp.int32),
            pltpu.VMEM_SHARED((16 * NPOOL * BPW, EMB), jnp.float32),
            pltpu.VMEM((CHUNK, EMB), jnp.float32),
            pltpu.SemaphoreType.DMA,
        ],
    )
    return f(idx_all, dest, zeros, ta, tu, tt, tal, tg, tn, tc)


BB = 512


def _hist(x, bnds, pw, mask_first):
    idx = jnp.zeros(x.shape, jnp.int32)
    for b in bnds:
        idx = idx + (x >= float(b)).astype(jnp.int32)
    cols = lax.broadcasted_iota(jnp.int32, (x.shape[0], pw), 1)
    h = jnp.zeros((x.shape[0], pw), jnp.float32)
    for l in range(x.shape[1]):
        h = h + (idx[:, l:l + 1] == cols).astype(jnp.float32)
    if mask_first:
        h = h * (cols != 0).astype(jnp.float32)
    return h


def _tc_body(name_e, can_e, s_a, s_u, s_t, s_al, s_g,
             ia, iu, it, ial, ig,
             collab, ns, na, nb_,
             dur, apop, fol, tpop,
             row0s, t_cl, t_ns, t_na, t_nb, t_dur, t_ap, t_fol, t_tp,
             U, V, cb, W1, b1, W2, b2, W3, b3, out):
    f32 = jnp.float32

    def pooled(sref, iref, k):
        n0 = jnp.sum((iref[...] == 0).astype(f32), axis=1, keepdims=True)
        cnt = L - n0
        num = sref[...] - n0 * row0s[k:k + 1, :]
        return jnp.where(cnt > 0.5, num / jnp.maximum(cnt, 1.0), 0.0)

    cl_cols = lax.broadcasted_iota(jnp.int32, (collab.shape[0], 8), 1)
    oh_cl = (collab[...] == cl_cols).astype(f32)
    e_cl = jnp.dot(oh_cl, t_cl[...], preferred_element_type=f32)
    e_ns = jnp.dot(_hist(ns[...], NB, 8, False), t_ns[...], preferred_element_type=f32)
    e_na = jnp.dot(_hist(na[...], NB, 8, False), t_na[...], preferred_element_type=f32)
    e_nb = jnp.dot(_hist(nb_[...], NB, 8, False), t_nb[...], preferred_element_type=f32)

    h_dur = _hist(dur[...], DUR_B, 104, True)
    cnt_d = jnp.sum(h_dur, axis=1, keepdims=True)
    e_dur = jnp.dot(h_dur, t_dur[...], preferred_element_type=f32) / jnp.maximum(cnt_d, 1e-9)

    e_ap = jnp.dot(_hist(apop[...], NB, 8, False), t_ap[...], preferred_element_type=f32) * (1.0 / L)
    e_fol = jnp.dot(_hist(fol[...], FOL_B, 16, False), t_fol[...], preferred_element_type=f32) * (1.0 / L)
    e_tp = jnp.dot(_hist(tpop[...], TPOP_B, 16, False), t_tp[...], preferred_element_type=f32) * (1.0 / L)

    x0 = jnp.concatenate([
        name_e[...], e_cl, can_e[...], e_ns, e_na, e_nb,
        pooled(s_a, ia, 0), pooled(s_u, iu, 1), pooled(s_t, it, 2),
        e_dur, pooled(s_al, ial, 3),
        e_ap, e_fol, e_tp, pooled(s_g, ig, 4),
    ], axis=1)

    c1 = jnp.dot(x0, U[...], preferred_element_type=f32)
    c2 = jnp.dot(c1, V[...], preferred_element_type=f32)
    x = x0 * (c2 + cb[0:1, :]) + x0
    h = jnp.maximum(jnp.dot(x, W1[...], preferred_element_type=f32) + b1[0:1, :], 0.0)
    h = jnp.maximum(jnp.dot(h, W2[...], preferred_element_type=f32) + b2[0:1, :], 0.0)
    h = jnp.dot(h, W3[...], preferred_element_type=f32) + b3[0:1, :]
    nrm = lax.rsqrt(jnp.maximum(jnp.sum(h * h, axis=1, keepdims=True), 1e-12))
    out[...] = h * nrm


def _pad_rows(a, n):
    return jnp.pad(a, ((0, n - a.shape[0]), (0, 0)))


def kernel(name, collaborative, track_uri_can, n_songs_pl, num_artists_pl,
           num_albums_pl, artist_name_pl, track_uri_pl, track_name_pl,
           duration_ms_songs_pl, album_name_pl, artist_pop_pl,
           artists_followers_pl, track_pop_pl, artist_genres_pl, params):
    p = params
    i32 = jnp.int32
    f32 = jnp.float32

    ia = artist_name_pl.astype(i32)
    iu = track_uri_pl.astype(i32)
    it = track_name_pl.astype(i32)
    ial = album_name_pl.astype(i32)
    ig = artist_genres_pl.astype(i32)
    iname = name.astype(i32)
    ican = track_uri_can.astype(i32)

    def pack_pool(a):
        return a.reshape(NW, NCHUNK, CHUNK)

    idx_all = jnp.concatenate(
        [pack_pool(ia), pack_pool(iu), pack_pool(it), pack_pool(ial),
         pack_pool(ig), iname.reshape(NW, 1, CHUNK), ican.reshape(NW, 1, CHUNK)],
        axis=1)

    base = np.repeat(np.arange(BPW, dtype=np.int32), L).reshape(NCHUNK, CHUNK)
    one = np.concatenate([base + f * BPW for f in range(NPOOL)], axis=0)
    dest = jnp.asarray(
        np.stack([one + s * (NPOOL * BPW) for s in range(16)], axis=0))
    zeros = jnp.zeros((NPOOL * BPW, EMB), f32)

    s_a, s_u, s_t, s_al, s_g, name_e, can_e = _sc_gather(
        idx_all, dest, zeros,
        p['artist_name_emb'], p['track_uri_pl_emb'], p['track_name_emb'],
        p['album_name_emb'], p['genres_emb'], p['name_emb'],
        p['track_uri_can_emb'])

    row0s = _pad_rows(jnp.stack([
        p['artist_name_emb'][0], p['track_uri_pl_emb'][0],
        p['track_name_emb'][0], p['album_name_emb'][0],
        p['genres_emb'][0]]), 8)

    d0 = 15 * EMB
    U = jnp.pad(p['cross_U'], ((0, 0), (0, 28)))
    V = jnp.pad(p['cross_V'], ((0, 28), (0, 0)))
    cb = _pad_rows(p['cross_b'].reshape(1, d0), 8)
    b1 = _pad_rows(p['b1'].reshape(1, -1), 8)
    b2 = _pad_rows(p['b2'].reshape(1, -1), 8)
    b3 = _pad_rows(p['b3'].reshape(1, -1), 8)

    grid = (B // BB,)

    def bspec(shape2):
        return pl.BlockSpec((BB, shape2), lambda i: (i, 0))

    def wspec(shape):
        return pl.BlockSpec(shape, lambda i: tuple(0 for _ in shape))

    in_specs = (
        [bspec(EMB)] * 7
        + [bspec(L)] * 5
        + [bspec(1)] * 4
        + [bspec(L)] * 4
        + [wspec((8, EMB)), wspec((8, EMB)), wspec((8, EMB)), wspec((8, EMB)),
           wspec((8, EMB)), wspec((104, EMB)), wspec((8, EMB)),
           wspec((16, EMB)), wspec((16, EMB)),
           wspec((d0, 128)), wspec((128, d0)), wspec((8, d0)),
           wspec((d0, 512)), wspec((8, 512)),
           wspec((512, 256)), wspec((8, 256)),
           wspec((256, 128)), wspec((8, 128))]
    )

    out = pl.pallas_call(
        _tc_body,
        grid=grid,
        in_specs=in_specs,
        out_specs=pl.BlockSpec((BB, EMB), lambda i: (i, 0)),
        out_shape=jax.ShapeDtypeStruct((B, EMB), f32),
    )(
        name_e, can_e, s_a, s_u, s_t, s_al, s_g,
        ia, iu, it, ial, ig,
        collaborative.astype(i32).reshape(B, 1),
        n_songs_pl.astype(f32).reshape(B, 1),
        num_artists_pl.astype(f32).reshape(B, 1),
        num_albums_pl.astype(f32).reshape(B, 1),
        duration_ms_songs_pl.astype(f32),
        artist_pop_pl.astype(f32),
        artists_followers_pl.astype(f32),
        track_pop_pl.astype(f32),
        row0s,
        _pad_rows(p['collab_emb'], 8),
        _pad_rows(p['n_songs_emb'], 8),
        _pad_rows(p['n_artists_emb'], 8),
        _pad_rows(p['n_albums_emb'], 8),
        _pad_rows(p['duration_emb'], 104),
        _pad_rows(p['artist_pop_emb'], 8),
        _pad_rows(p['followers_emb'], 16),
        _pad_rows(p['track_pop_emb'], 16),
        U, V, cb, p['W1'], b1, p['W2'], b2, p['W3'], b3,
    )
    return out


# speedup vs baseline: 21.1067x; 1.1500x over previous
import functools

import numpy as np
import jax
import jax.numpy as jnp
from jax import lax
from jax.experimental import pallas as pl
from jax.experimental.pallas import tpu as pltpu
from jax.experimental.pallas import tpu_sc as plsc

B = 4096
L = 20
EMB = 128
NW = 32
BPW = B // NW
CHUNK = 64
NCHUNK = (BPW * L) // CHUNK
NSROW = BPW // CHUNK
NPOOL = 5
WROWS = NPOOL * NCHUNK + 2 * NSROW + 4

NB = np.linspace(1.0, 5.0, 5).astype(np.float32)
DUR_B = np.linspace(-1.0, 20744575.0, 100).astype(np.float32)
FOL_B = np.linspace(0.0, 94437255.0, 10).astype(np.float32)
TPOP_B = np.linspace(-1.0, 96.0, 10).astype(np.float32)


def _sc_body(idx_hbm, dest_hbm, zeros_hbm,
             ta, tu, tt, tal, tg, tn, tc,
             o_a, o_u, o_t, o_al, o_g, o_n, o_c,
             idx_v, idx_s, dest_v, acc, g0, g1, sem0, sem1):
    c = lax.axis_index("c")
    s = lax.axis_index("s")
    w = s * 2 + c
    sbase = s * (NPOOL * BPW)
    wrows = WROWS
    pltpu.sync_copy(dest_hbm.at[s], dest_v)
    pltpu.sync_copy(idx_hbm.at[pl.ds(w * wrows + NPOOL * NCHUNK, 2 * NSROW)],
                    idx_s)
    pltpu.sync_copy(zeros_hbm, acc.at[pl.ds(sbase, NPOOL * BPW)])

    def drain(buf, sem):
        pltpu.make_async_copy(zeros_hbm.at[pl.ds(0, CHUNK)], buf, sem).wait()

    tables = [ta, tu, tt, tal, tg]
    for f in range(NPOOL):
        tbl = tables[f]
        r0 = f * NCHUNK
        pltpu.sync_copy(idx_hbm.at[pl.ds(w * wrows + r0, NCHUNK)], idx_v)
        pltpu.async_copy(tbl.at[idx_v.at[0]], g0, sem0)
        pltpu.async_copy(tbl.at[idx_v.at[1]], g1, sem1)

        def body(j, _, tbl=tbl, r0=r0):
            r = 2 * j
            drain(g0, sem0)
            pltpu.sync_copy(g0, acc.at[dest_v.at[r0 + r]], add=True)
            pltpu.async_copy(tbl.at[idx_v.at[r + 2]], g0, sem0)
            drain(g1, sem1)
            pltpu.sync_copy(g1, acc.at[dest_v.at[r0 + r + 1]], add=True)
            pltpu.async_copy(tbl.at[idx_v.at[r + 3]], g1, sem1)
            return 0

        lax.fori_loop(0, NCHUNK // 2 - 1, body, 0)
        drain(g0, sem0)
        pltpu.sync_copy(g0, acc.at[dest_v.at[r0 + NCHUNK - 2]], add=True)
        drain(g1, sem1)
        pltpu.sync_copy(g1, acc.at[dest_v.at[r0 + NCHUNK - 1]], add=True)

    row0 = w * BPW

    pltpu.async_copy(tn.at[idx_s.at[0]], g0, sem0)
    pltpu.async_copy(tn.at[idx_s.at[1]], g1, sem1)

    outs = [o_a, o_u, o_t, o_al, o_g]
    for f in range(NPOOL):
        pltpu.sync_copy(acc.at[pl.ds(sbase + f * BPW, BPW)],
                        outs[f].at[pl.ds(row0, BPW)])

    drain(g0, sem0)
    pltpu.sync_copy(g0, o_n.at[pl.ds(row0, CHUNK)])
    pltpu.async_copy(tc.at[idx_s.at[2]], g0, sem0)
    drain(g1, sem1)
    pltpu.sync_copy(g1, o_n.at[pl.ds(row0 + CHUNK, CHUNK)])
    pltpu.async_copy(tc.at[idx_s.at[3]], g1, sem1)
    drain(g0, sem0)
    pltpu.sync_copy(g0, o_c.at[pl.ds(row0, CHUNK)])
    drain(g1, sem1)
    pltpu.sync_copy(g1, o_c.at[pl.ds(row0 + CHUNK, CHUNK)])


def _sc_gather(idx_all, dest, zeros, ta, tu, tt, tal, tg, tn, tc):
    out = tuple(jax.ShapeDtypeStruct((B, EMB), jnp.float32) for _ in range(7))
    mesh = plsc.VectorSubcoreMesh(core_axis_name="c", subcore_axis_name="s")
    f = pl.kernel(
        _sc_body,
        out_type=out,
        mesh=mesh,
        scratch_types=[
            pltpu.VMEM((NCHUNK, CHUNK), jnp.int32),
            pltpu.VMEM((2 * NSROW, CHUNK), jnp.int32),
            pltpu.VMEM((NPOOL * NCHUNK, CHUNK), jnp.int32),
            pltpu.VMEM_SHARED((16 * NPOOL * BPW, EMB), jnp.float32),
            pltpu.VMEM((CHUNK, EMB), jnp.float32),
            pltpu.VMEM((CHUNK, EMB), jnp.float32),
            pltpu.SemaphoreType.DMA,
            pltpu.SemaphoreType.DMA,
        ],
    )
    return f(idx_all, dest, zeros, ta, tu, tt, tal, tg, tn, tc)


BB = 512


def _hist(x, bnds, pw, mask_first):
    idx = jnp.zeros(x.shape, jnp.int32)
    for b in bnds:
        idx = idx + (x >= float(b)).astype(jnp.int32)
    cols = lax.broadcasted_iota(jnp.int32, (x.shape[0], pw), 1)
    h = jnp.zeros((x.shape[0], pw), jnp.float32)
    for l in range(x.shape[1]):
        h = h + (idx[:, l:l + 1] == cols).astype(jnp.float32)
    if mask_first:
        h = h * (cols != 0).astype(jnp.float32)
    return h


def _tc_body(name_e, can_e, s_a, s_u, s_t, s_al, s_g,
             ia, iu, it, ial, ig,
             collab, ns, na, nb_,
             dur, apop, fol, tpop,
             row0s, t_cl, t_ns, t_na, t_nb, t_dur, t_ap, t_fol, t_tp,
             U, V, cb, W1, b1, W2, b2, W3, b3, out):
    f32 = jnp.float32

    def pooled(sref, iref, k):
        n0 = jnp.sum((iref[...] == 0).astype(f32), axis=1, keepdims=True)
        cnt = L - n0
        num = sref[...] - n0 * row0s[k:k + 1, :]
        return jnp.where(cnt > 0.5, num / jnp.maximum(cnt, 1.0), 0.0)

    cl_cols = lax.broadcasted_iota(jnp.int32, (collab.shape[0], 8), 1)
    oh_cl = (collab[...] == cl_cols).astype(f32)
    e_cl = jnp.dot(oh_cl, t_cl[...], preferred_element_type=f32)
    e_ns = jnp.dot(_hist(ns[...], NB, 8, False), t_ns[...], preferred_element_type=f32)
    e_na = jnp.dot(_hist(na[...], NB, 8, False), t_na[...], preferred_element_type=f32)
    e_nb = jnp.dot(_hist(nb_[...], NB, 8, False), t_nb[...], preferred_element_type=f32)

    h_dur = _hist(dur[...], DUR_B, 104, True)
    cnt_d = jnp.sum(h_dur, axis=1, keepdims=True)
    e_dur = jnp.dot(h_dur, t_dur[...], preferred_element_type=f32) / jnp.maximum(cnt_d, 1e-9)

    e_ap = jnp.dot(_hist(apop[...], NB, 8, False), t_ap[...], preferred_element_type=f32) * (1.0 / L)
    e_fol = jnp.dot(_hist(fol[...], FOL_B, 16, False), t_fol[...], preferred_element_type=f32) * (1.0 / L)
    e_tp = jnp.dot(_hist(tpop[...], TPOP_B, 16, False), t_tp[...], preferred_element_type=f32) * (1.0 / L)

    x0 = jnp.concatenate([
        name_e[...], e_cl, can_e[...], e_ns, e_na, e_nb,
        pooled(s_a, ia, 0), pooled(s_u, iu, 1), pooled(s_t, it, 2),
        e_dur, pooled(s_al, ial, 3),
        e_ap, e_fol, e_tp, pooled(s_g, ig, 4),
    ], axis=1)

    c1 = jnp.dot(x0, U[...], preferred_element_type=f32)
    c2 = jnp.dot(c1, V[...], preferred_element_type=f32)
    x = x0 * (c2 + cb[0:1, :]) + x0
    h = jnp.maximum(jnp.dot(x, W1[...], preferred_element_type=f32) + b1[0:1, :], 0.0)
    h = jnp.maximum(jnp.dot(h, W2[...], preferred_element_type=f32) + b2[0:1, :], 0.0)
    h = jnp.dot(h, W3[...], preferred_element_type=f32) + b3[0:1, :]
    nrm = lax.rsqrt(jnp.maximum(jnp.sum(h * h, axis=1, keepdims=True), 1e-12))
    out[...] = h * nrm


def _pad_rows(a, n):
    return jnp.pad(a, ((0, n - a.shape[0]), (0, 0)))


def kernel(name, collaborative, track_uri_can, n_songs_pl, num_artists_pl,
           num_albums_pl, artist_name_pl, track_uri_pl, track_name_pl,
           duration_ms_songs_pl, album_name_pl, artist_pop_pl,
           artists_followers_pl, track_pop_pl, artist_genres_pl, params):
    p = params
    i32 = jnp.int32
    f32 = jnp.float32

    ia = artist_name_pl.astype(i32)
    iu = track_uri_pl.astype(i32)
    it = track_name_pl.astype(i32)
    ial = album_name_pl.astype(i32)
    ig = artist_genres_pl.astype(i32)
    iname = name.astype(i32)
    ican = track_uri_can.astype(i32)

    def pack_pool(a):
        return a.reshape(NW, NCHUNK, CHUNK)

    idx_all = jnp.concatenate(
        [pack_pool(ia), pack_pool(iu), pack_pool(it), pack_pool(ial),
         pack_pool(ig), iname.reshape(NW, NSROW, CHUNK),
         ican.reshape(NW, NSROW, CHUNK),
         jnp.zeros((NW, 4, CHUNK), i32)],
        axis=1).reshape(NW * WROWS, CHUNK)

    base = np.repeat(np.arange(BPW, dtype=np.int32), L).reshape(NCHUNK, CHUNK)
    one = np.concatenate([base + f * BPW for f in range(NPOOL)], axis=0)
    dest = jnp.asarray(
        np.stack([one + s * (NPOOL * BPW) for s in range(16)], axis=0))
    zeros = jnp.zeros((NPOOL * BPW, EMB), f32)

    s_a, s_u, s_t, s_al, s_g, name_e, can_e = _sc_gather(
        idx_all, dest, zeros,
        p['artist_name_emb'], p['track_uri_pl_emb'], p['track_name_emb'],
        p['album_name_emb'], p['genres_emb'], p['name_emb'],
        p['track_uri_can_emb'])

    row0s = _pad_rows(jnp.stack([
        p['artist_name_emb'][0], p['track_uri_pl_emb'][0],
        p['track_name_emb'][0], p['album_name_emb'][0],
        p['genres_emb'][0]]), 8)

    d0 = 15 * EMB
    U = jnp.pad(p['cross_U'], ((0, 0), (0, 28)))
    V = jnp.pad(p['cross_V'], ((0, 28), (0, 0)))
    cb = _pad_rows(p['cross_b'].reshape(1, d0), 8)
    b1 = _pad_rows(p['b1'].reshape(1, -1), 8)
    b2 = _pad_rows(p['b2'].reshape(1, -1), 8)
    b3 = _pad_rows(p['b3'].reshape(1, -1), 8)

    grid = (B // BB,)

    def bspec(shape2):
        return pl.BlockSpec((BB, shape2), lambda i: (i, 0))

    def wspec(shape):
        return pl.BlockSpec(shape, lambda i: tuple(0 for _ in shape))

    in_specs = (
        [bspec(EMB)] * 7
        + [bspec(L)] * 5
        + [bspec(1)] * 4
        + [bspec(L)] * 4
        + [wspec((8, EMB)), wspec((8, EMB)), wspec((8, EMB)), wspec((8, EMB)),
           wspec((8, EMB)), wspec((104, EMB)), wspec((8, EMB)),
           wspec((16, EMB)), wspec((16, EMB)),
           wspec((d0, 128)), wspec((128, d0)), wspec((8, d0)),
           wspec((d0, 512)), wspec((8, 512)),
           wspec((512, 256)), wspec((8, 256)),
           wspec((256, 128)), wspec((8, 128))]
    )

    out = pl.pallas_call(
        _tc_body,
        grid=grid,
        in_specs=in_specs,
        out_specs=pl.BlockSpec((BB, EMB), lambda i: (i, 0)),
        out_shape=jax.ShapeDtypeStruct((B, EMB), f32),
    )(
        name_e, can_e, s_a, s_u, s_t, s_al, s_g,
        ia, iu, it, ial, ig,
        collaborative.astype(i32).reshape(B, 1),
        n_songs_pl.astype(f32).reshape(B, 1),
        num_artists_pl.astype(f32).reshape(B, 1),
        num_albums_pl.astype(f32).reshape(B, 1),
        duration_ms_songs_pl.astype(f32),
        artist_pop_pl.astype(f32),
        artists_followers_pl.astype(f32),
        track_pop_pl.astype(f32),
        row0s,
        _pad_rows(p['collab_emb'], 8),
        _pad_rows(p['n_songs_emb'], 8),
        _pad_rows(p['n_artists_emb'], 8),
        _pad_rows(p['n_albums_emb'], 8),
        _pad_rows(p['duration_emb'], 104),
        _pad_rows(p['artist_pop_emb'], 8),
        _pad_rows(p['followers_emb'], 16),
        _pad_rows(p['track_pop_emb'], 16),
        U, V, cb, p['W1'], b1, p['W2'], b2, p['W3'], b3,
    )
    return out


# speedup vs baseline: 22.0218x; 1.0434x over previous
import functools

import numpy as np
import jax
import jax.numpy as jnp
from jax import lax
from jax.experimental import pallas as pl
from jax.experimental.pallas import tpu as pltpu
from jax.experimental.pallas import tpu_sc as plsc

B = 4096
L = 20
EMB = 128
NW = 32
NPOOL = 5
NSPLIT = 2
BS = B // NSPLIT

NB = np.linspace(1.0, 5.0, 5).astype(np.float32)
DUR_B = np.linspace(-1.0, 20744575.0, 100).astype(np.float32)
FOL_B = np.linspace(0.0, 94437255.0, 10).astype(np.float32)
TPOP_B = np.linspace(-1.0, 96.0, 10).astype(np.float32)


def _sc_dims(bsz):
    bpw = bsz // NW
    chunk = min(64, bpw)
    nchunk = (bpw * L) // chunk
    nsrow = bpw // chunk
    stride = nchunk + (-nchunk) % 8
    srows = 2 * nsrow + (-2 * nsrow) % 8
    wrows = NPOOL * stride + srows
    return bpw, chunk, nchunk, nsrow, stride, srows, wrows


def _sc_body(dims,
             idx_hbm, dest_hbm, zeros_hbm,
             ta, tu, tt, tal, tg, tn, tc,
             o_a, o_u, o_t, o_al, o_g, o_n, o_c,
             idx_v, idx_s, dest_v, acc, g0, g1, sem0, sem1):
    bpw, chunk, nchunk, nsrow, stride, srows, wrows = dims
    c = lax.axis_index("c")
    s = lax.axis_index("s")
    w = s * 2 + c
    sbase = s * (NPOOL * bpw)
    pltpu.sync_copy(dest_hbm.at[s], dest_v)
    pltpu.sync_copy(idx_hbm.at[pl.ds(w * wrows + NPOOL * stride, srows)],
                    idx_s)
    pltpu.sync_copy(zeros_hbm, acc.at[pl.ds(sbase, NPOOL * bpw)])

    def drain(buf, sem):
        pltpu.make_async_copy(zeros_hbm.at[pl.ds(0, chunk)], buf, sem).wait()

    tables = [ta, tu, tt, tal, tg]
    for f in range(NPOOL):
        tbl = tables[f]
        r0 = f * nchunk
        pltpu.sync_copy(idx_hbm.at[pl.ds(w * wrows + f * stride, stride)],
                        idx_v)
        pltpu.async_copy(tbl.at[idx_v.at[0]], g0, sem0)
        pltpu.async_copy(tbl.at[idx_v.at[1]], g1, sem1)

        def body(j, _, tbl=tbl, r0=r0):
            r = 2 * j
            drain(g0, sem0)
            pltpu.sync_copy(g0, acc.at[dest_v.at[r0 + r]], add=True)
            pltpu.async_copy(tbl.at[idx_v.at[r + 2]], g0, sem0)
            drain(g1, sem1)
            pltpu.sync_copy(g1, acc.at[dest_v.at[r0 + r + 1]], add=True)
            pltpu.async_copy(tbl.at[idx_v.at[r + 3]], g1, sem1)
            return 0

        lax.fori_loop(0, nchunk // 2 - 1, body, 0)
        drain(g0, sem0)
        pltpu.sync_copy(g0, acc.at[dest_v.at[r0 + nchunk - 2]], add=True)
        drain(g1, sem1)
        pltpu.sync_copy(g1, acc.at[dest_v.at[r0 + nchunk - 1]], add=True)

    row0 = w * bpw
    bufs = [(g0, sem0), (g1, sem1)]
    seq = ([(k, o_n, k * chunk, tn) for k in range(nsrow)]
           + [(nsrow + k, o_c, k * chunk, tc) for k in range(nsrow)])
    pend = []
    for i, (r, oref, off, tbl) in enumerate(seq):
        if len(pend) == 2:
            po, poff, pbuf, psem = pend.pop(0)
            drain(pbuf, psem)
            pltpu.sync_copy(pbuf, po.at[pl.ds(row0 + poff, chunk)])
        buf, sem = bufs[i % 2]
        pltpu.async_copy(tbl.at[idx_s.at[r]], buf, sem)
        pend.append((oref, off, buf, sem))

    outs = [o_a, o_u, o_t, o_al, o_g]
    for f in range(NPOOL):
        pltpu.sync_copy(acc.at[pl.ds(sbase + f * bpw, bpw)],
                        outs[f].at[pl.ds(row0, bpw)])

    for po, poff, pbuf, psem in pend:
        drain(pbuf, psem)
        pltpu.sync_copy(pbuf, po.at[pl.ds(row0 + poff, chunk)])


@functools.cache
def _sc_kernel(bsz):
    dims = _sc_dims(bsz)
    bpw, chunk, nchunk, nsrow, stride, srows, wrows = dims
    ndest = NPOOL * nchunk + (-NPOOL * nchunk) % 8
    out = tuple(jax.ShapeDtypeStruct((bsz, EMB), jnp.float32)
                for _ in range(7))
    mesh = plsc.VectorSubcoreMesh(core_axis_name="c", subcore_axis_name="s")
    return pl.kernel(
        functools.partial(_sc_body, dims),
        out_type=out,
        mesh=mesh,
        scratch_types=[
            pltpu.VMEM((stride, chunk), jnp.int32),
            pltpu.VMEM((srows, chunk), jnp.int32),
            pltpu.VMEM((ndest, chunk), jnp.int32),
            pltpu.VMEM_SHARED((16 * NPOOL * bpw, EMB), jnp.float32),
            pltpu.VMEM((chunk, EMB), jnp.float32),
            pltpu.VMEM((chunk, EMB), jnp.float32),
            pltpu.SemaphoreType.DMA,
            pltpu.SemaphoreType.DMA,
        ],
    )


BB = 512


def _hist(x, bnds, pw, mask_first):
    idx = jnp.zeros(x.shape, jnp.int32)
    for b in bnds:
        idx = idx + (x >= float(b)).astype(jnp.int32)
    cols = lax.broadcasted_iota(jnp.int32, (x.shape[0], pw), 1)
    h = jnp.zeros((x.shape[0], pw), jnp.float32)
    for l in range(x.shape[1]):
        h = h + (idx[:, l:l + 1] == cols).astype(jnp.float32)
    if mask_first:
        h = h * (cols != 0).astype(jnp.float32)
    return h


def _tc_body(name_e, can_e, s_a, s_u, s_t, s_al, s_g,
             ia, iu, it, ial, ig,
             collab, ns, na, nb_,
             dur, apop, fol, tpop,
             row0s, t_cl, t_ns, t_na, t_nb, t_dur, t_ap, t_fol, t_tp,
             U, V, cb, W1, b1, W2, b2, W3, b3, out):
    f32 = jnp.float32

    def pooled(sref, iref, k):
        n0 = jnp.sum((iref[...] == 0).astype(f32), axis=1, keepdims=True)
        cnt = L - n0
        num = sref[...] - n0 * row0s[k:k + 1, :]
        return jnp.where(cnt > 0.5, num / jnp.maximum(cnt, 1.0), 0.0)

    cl_cols = lax.broadcasted_iota(jnp.int32, (collab.shape[0], 8), 1)
    oh_cl = (collab[...] == cl_cols).astype(f32)
    e_cl = jnp.dot(oh_cl, t_cl[...], preferred_element_type=f32)
    e_ns = jnp.dot(_hist(ns[...], NB, 8, False), t_ns[...], preferred_element_type=f32)
    e_na = jnp.dot(_hist(na[...], NB, 8, False), t_na[...], preferred_element_type=f32)
    e_nb = jnp.dot(_hist(nb_[...], NB, 8, False), t_nb[...], preferred_element_type=f32)

    h_dur = _hist(dur[...], DUR_B, 104, True)
    cnt_d = jnp.sum(h_dur, axis=1, keepdims=True)
    e_dur = jnp.dot(h_dur, t_dur[...], preferred_element_type=f32) / jnp.maximum(cnt_d, 1e-9)

    e_ap = jnp.dot(_hist(apop[...], NB, 8, False), t_ap[...], preferred_element_type=f32) * (1.0 / L)
    e_fol = jnp.dot(_hist(fol[...], FOL_B, 16, False), t_fol[...], preferred_element_type=f32) * (1.0 / L)
    e_tp = jnp.dot(_hist(tpop[...], TPOP_B, 16, False), t_tp[...], preferred_element_type=f32) * (1.0 / L)

    x0 = jnp.concatenate([
        name_e[...], e_cl, can_e[...], e_ns, e_na, e_nb,
        pooled(s_a, ia, 0), pooled(s_u, iu, 1), pooled(s_t, it, 2),
        e_dur, pooled(s_al, ial, 3),
        e_ap, e_fol, e_tp, pooled(s_g, ig, 4),
    ], axis=1)

    c1 = jnp.dot(x0, U[...], preferred_element_type=f32)
    c2 = jnp.dot(c1, V[...], preferred_element_type=f32)
    x = x0 * (c2 + cb[0:1, :]) + x0
    h = jnp.maximum(jnp.dot(x, W1[...], preferred_element_type=f32) + b1[0:1, :], 0.0)
    h = jnp.maximum(jnp.dot(h, W2[...], preferred_element_type=f32) + b2[0:1, :], 0.0)
    h = jnp.dot(h, W3[...], preferred_element_type=f32) + b3[0:1, :]
    nrm = lax.rsqrt(jnp.maximum(jnp.sum(h * h, axis=1, keepdims=True), 1e-12))
    out[...] = h * nrm


def _pad_rows(a, n):
    return jnp.pad(a, ((0, n - a.shape[0]), (0, 0)))


def kernel(name, collaborative, track_uri_can, n_songs_pl, num_artists_pl,
           num_albums_pl, artist_name_pl, track_uri_pl, track_name_pl,
           duration_ms_songs_pl, album_name_pl, artist_pop_pl,
           artists_followers_pl, track_pop_pl, artist_genres_pl, params):
    p = params
    i32 = jnp.int32
    f32 = jnp.float32
    bpw, chunk, nchunk, nsrow, stride, srows, wrows = _sc_dims(BS)

    ia = artist_name_pl.astype(i32)
    iu = track_uri_pl.astype(i32)
    it = track_name_pl.astype(i32)
    ial = album_name_pl.astype(i32)
    ig = artist_genres_pl.astype(i32)
    iname = name.astype(i32)
    ican = track_uri_can.astype(i32)

    base = np.repeat(np.arange(bpw, dtype=np.int32), L).reshape(nchunk, chunk)
    one = np.concatenate([base + f * bpw for f in range(NPOOL)], axis=0)
    ndest = NPOOL * nchunk + (-NPOOL * nchunk) % 8
    one = np.concatenate(
        [one, np.zeros((ndest - NPOOL * nchunk, chunk), np.int32)], axis=0)
    dest = jnp.asarray(
        np.stack([one + s * (NPOOL * bpw) for s in range(16)], axis=0))
    zeros = jnp.zeros((NPOOL * bpw, EMB), f32)

    sc = _sc_kernel(BS)
    spad = srows - 2 * nsrow
    fpad = stride - nchunk

    sc_outs = []
    for h in range(NSPLIT):
        sl = slice(h * BS, (h + 1) * BS)

        def pack_pool(a):
            blk = a[sl].reshape(NW, nchunk, chunk)
            return jnp.pad(blk, ((0, 0), (0, fpad), (0, 0)))

        idx_all = jnp.concatenate(
            [pack_pool(ia), pack_pool(iu), pack_pool(it), pack_pool(ial),
             pack_pool(ig), iname[sl].reshape(NW, nsrow, chunk),
             ican[sl].reshape(NW, nsrow, chunk),
             jnp.zeros((NW, spad, chunk), i32)],
            axis=1).reshape(NW * wrows, chunk)

        sc_outs.append(sc(
            idx_all, dest, zeros,
            p['artist_name_emb'], p['track_uri_pl_emb'], p['track_name_emb'],
            p['album_name_emb'], p['genres_emb'], p['name_emb'],
            p['track_uri_can_emb']))

    row0s = _pad_rows(jnp.stack([
        p['artist_name_emb'][0], p['track_uri_pl_emb'][0],
        p['track_name_emb'][0], p['album_name_emb'][0],
        p['genres_emb'][0]]), 8)

    d0 = 15 * EMB
    U = jnp.pad(p['cross_U'], ((0, 0), (0, 28)))
    V = jnp.pad(p['cross_V'], ((0, 28), (0, 0)))
    cb = _pad_rows(p['cross_b'].reshape(1, d0), 8)
    b1 = _pad_rows(p['b1'].reshape(1, -1), 8)
    b2 = _pad_rows(p['b2'].reshape(1, -1), 8)
    b3 = _pad_rows(p['b3'].reshape(1, -1), 8)

    def bspec(shape2):
        return pl.BlockSpec((BB, shape2), lambda i: (i, 0))

    def wspec(shape):
        return pl.BlockSpec(shape, lambda i: tuple(0 for _ in shape))

    in_specs = (
        [bspec(EMB)] * 7
        + [bspec(L)] * 5
        + [bspec(1)] * 4
        + [bspec(L)] * 4
        + [wspec((8, EMB)), wspec((8, EMB)), wspec((8, EMB)), wspec((8, EMB)),
           wspec((8, EMB)), wspec((104, EMB)), wspec((8, EMB)),
           wspec((16, EMB)), wspec((16, EMB)),
           wspec((d0, 128)), wspec((128, d0)), wspec((8, d0)),
           wspec((d0, 512)), wspec((8, 512)),
           wspec((512, 256)), wspec((8, 256)),
           wspec((256, 128)), wspec((8, 128))]
    )

    outs = []
    for h in range(NSPLIT):
        sl = slice(h * BS, (h + 1) * BS)
        s_a, s_u, s_t, s_al, s_g, name_e, can_e = sc_outs[h]
        outs.append(pl.pallas_call(
            _tc_body,
            grid=(BS // BB,),
            in_specs=in_specs,
            out_specs=pl.BlockSpec((BB, EMB), lambda i: (i, 0)),
            out_shape=jax.ShapeDtypeStruct((BS, EMB), f32),
        )(
            name_e, can_e, s_a, s_u, s_t, s_al, s_g,
            ia[sl], iu[sl], it[sl], ial[sl], ig[sl],
            collaborative.astype(i32)[sl].reshape(BS, 1),
            n_songs_pl.astype(f32)[sl].reshape(BS, 1),
            num_artists_pl.astype(f32)[sl].reshape(BS, 1),
            num_albums_pl.astype(f32)[sl].reshape(BS, 1),
            duration_ms_songs_pl.astype(f32)[sl],
            artist_pop_pl.astype(f32)[sl],
            artists_followers_pl.astype(f32)[sl],
            track_pop_pl.astype(f32)[sl],
            row0s,
            _pad_rows(p['collab_emb'], 8),
            _pad_rows(p['n_songs_emb'], 8),
            _pad_rows(p['n_artists_emb'], 8),
            _pad_rows(p['n_albums_emb'], 8),
            _pad_rows(p['duration_emb'], 104),
            _pad_rows(p['artist_pop_emb'], 8),
            _pad_rows(p['followers_emb'], 16),
            _pad_rows(p['track_pop_emb'], 16),
            U, V, cb, p['W1'], b1, p['W2'], b2, p['W3'], b3,
        ))
    return jnp.concatenate(outs, axis=0)


# speedup vs baseline: 23.6761x; 1.0751x over previous
import functools

import numpy as np
import jax
import jax.numpy as jnp
from jax import lax
from jax.experimental import pallas as pl
from jax.experimental.pallas import tpu as pltpu
from jax.experimental.pallas import tpu_sc as plsc

B = 4096
L = 20
EMB = 128
NW = 32
NPOOL = 5
NSPLIT = 2
BS = B // NSPLIT

BPW = BS // NW
PCH = 128
PNC = (BPW * L) // PCH
SCH = BPW
IBANK = PNC + 8 - PNC % 8 if PNC % 8 else PNC
NDEST = NPOOL * PNC + (-(NPOOL * PNC)) % 8

NB = np.linspace(1.0, 5.0, 5).astype(np.float32)
DUR_B = np.linspace(-1.0, 20744575.0, 100).astype(np.float32)
FOL_B = np.linspace(0.0, 94437255.0, 10).astype(np.float32)
TPOP_B = np.linspace(-1.0, 96.0, 10).astype(np.float32)


def _sc_body(i_a, i_u, i_t, i_al, i_g, i_n, i_c, dest_hbm, zeros_hbm,
             ta, tu, tt, tal, tg, tn, tc,
             o_a, o_u, o_t, o_al, o_g, o_n, o_c,
             idx_v, idx_s, dest_v, acc, g0, g1, sem0, sem1):
    c = lax.axis_index("c")
    s = lax.axis_index("s")
    w = s * 2 + c
    sbase = s * (NPOOL * BPW)
    row0 = w * BPW

    off = (w * PNC) % 8
    start = pl.multiple_of(w * PNC - off, 8)
    idx_ins = [i_a, i_u, i_t, i_al, i_g]
    for f in range(NPOOL):
        pltpu.sync_copy(idx_ins[f].at[pl.ds(start, IBANK)],
                        idx_v.at[pl.ds(f * IBANK, IBANK)])
    soff = w % 8
    sstart = pl.multiple_of(w - soff, 8)
    pltpu.sync_copy(i_n.at[pl.ds(sstart, 8)], idx_s.at[pl.ds(0, 8)])
    pltpu.sync_copy(i_c.at[pl.ds(sstart, 8)], idx_s.at[pl.ds(8, 8)])
    pltpu.sync_copy(dest_hbm.at[s], dest_v)
    pltpu.sync_copy(zeros_hbm, acc.at[pl.ds(sbase, NPOOL * BPW)])

    def drain(buf, sem, rows):
        pltpu.make_async_copy(zeros_hbm.at[pl.ds(0, rows)], buf, sem).wait()

    tables = [ta, tu, tt, tal, tg]
    for f in range(NPOOL):
        tbl = tables[f]
        b0 = f * IBANK
        pltpu.async_copy(tbl.at[idx_v.at[b0 + off]], g0, sem0)
        pltpu.async_copy(tbl.at[idx_v.at[b0 + off + 1]], g1, sem1)

        def body(j, _, tbl=tbl, b0=b0, f=f):
            r = 2 * j
            drain(g0, sem0, PCH)
            pltpu.sync_copy(g0, acc.at[dest_v.at[f * PNC + r]], add=True)
            pltpu.async_copy(tbl.at[idx_v.at[b0 + off + r + 2]], g0, sem0)
            drain(g1, sem1, PCH)
            pltpu.sync_copy(g1, acc.at[dest_v.at[f * PNC + r + 1]], add=True)
            pltpu.async_copy(tbl.at[idx_v.at[b0 + off + r + 3]], g1, sem1)
            return 0

        lax.fori_loop(0, PNC // 2 - 1, body, 0)
        drain(g0, sem0, PCH)
        pltpu.sync_copy(g0, acc.at[dest_v.at[f * PNC + PNC - 2]], add=True)
        drain(g1, sem1, PCH)
        pltpu.sync_copy(g1, acc.at[dest_v.at[f * PNC + PNC - 1]], add=True)

    g0s = g0.at[pl.ds(0, SCH)]
    g1s = g1.at[pl.ds(0, SCH)]
    pltpu.async_copy(tn.at[idx_s.at[soff]], g0s, sem0)
    pltpu.async_copy(tc.at[idx_s.at[8 + soff]], g1s, sem1)

    outs = [o_a, o_u, o_t, o_al, o_g]
    for f in range(NPOOL):
        pltpu.sync_copy(acc.at[pl.ds(sbase + f * BPW, BPW)],
                        outs[f].at[pl.ds(row0, BPW)])

    drain(g0s, sem0, SCH)
    pltpu.sync_copy(g0s, o_n.at[pl.ds(row0, SCH)])
    drain(g1s, sem1, SCH)
    pltpu.sync_copy(g1s, o_c.at[pl.ds(row0, SCH)])


@functools.cache
def _sc_kernel():
    out = tuple(jax.ShapeDtypeStruct((BS, EMB), jnp.float32)
                for _ in range(7))
    mesh = plsc.VectorSubcoreMesh(core_axis_name="c", subcore_axis_name="s")
    return pl.kernel(
        _sc_body,
        out_type=out,
        mesh=mesh,
        scratch_types=[
            pltpu.VMEM((NPOOL * IBANK, PCH), jnp.int32),
            pltpu.VMEM((16, SCH), jnp.int32),
            pltpu.VMEM((NDEST, PCH), jnp.int32),
            pltpu.VMEM_SHARED((16 * NPOOL * BPW, EMB), jnp.float32),
            pltpu.VMEM((PCH, EMB), jnp.float32),
            pltpu.VMEM((PCH, EMB), jnp.float32),
            pltpu.SemaphoreType.DMA,
            pltpu.SemaphoreType.DMA,
        ],
    )


BB = 1024

_T_CL, _T_NS, _T_NA, _T_NB = 0, 8, 16, 24
_T_DUR, _T_AP, _T_FOL, _T_TP = 32, 136, 144, 160
_T_ROWS = 176


def _hist(x, bnds, pw, mask_first):
    idx = jnp.zeros(x.shape, jnp.int32)
    for b in bnds:
        idx = idx + (x >= float(b)).astype(jnp.int32)
    cols = lax.broadcasted_iota(jnp.int32, (x.shape[0], pw), 1)
    h = jnp.zeros((x.shape[0], pw), jnp.float32)
    for l in range(x.shape[1]):
        h = h + (idx[:, l:l + 1] == cols).astype(jnp.float32)
    if mask_first:
        h = h * (cols != 0).astype(jnp.float32)
    return h


def _tc_body(name_e, can_e, s_a, s_u, s_t, s_al, s_g,
             ia, iu, it, ial, ig,
             collab, ns, na, nb_,
             dur, apop, fol, tpop,
             row0s, t_all_ref, U, V, b_all_ref, W1, W2, W3, out):
    f32 = jnp.float32
    t_all = t_all_ref[...]
    b_all = b_all_ref[...]

    def pooled(sref, iref, k):
        n0 = jnp.sum((iref[...] == 0).astype(f32), axis=1, keepdims=True)
        cnt = L - n0
        num = sref[...] - n0 * row0s[k:k + 1, :]
        return jnp.where(cnt > 0.5, num / jnp.maximum(cnt, 1.0), 0.0)

    def dot(a, b):
        return jnp.dot(a, b, preferred_element_type=f32)

    cl_cols = lax.broadcasted_iota(jnp.int32, (collab.shape[0], 8), 1)
    oh_cl = (collab[...] == cl_cols).astype(f32)
    e_cl = dot(oh_cl, t_all[_T_CL:_T_CL + 8])
    e_ns = dot(_hist(ns[...], NB, 8, False), t_all[_T_NS:_T_NS + 8])
    e_na = dot(_hist(na[...], NB, 8, False), t_all[_T_NA:_T_NA + 8])
    e_nb = dot(_hist(nb_[...], NB, 8, False), t_all[_T_NB:_T_NB + 8])

    h_dur = _hist(dur[...], DUR_B, 104, True)
    cnt_d = jnp.sum(h_dur, axis=1, keepdims=True)
    e_dur = dot(h_dur, t_all[_T_DUR:_T_DUR + 104]) / jnp.maximum(cnt_d, 1e-9)

    e_ap = dot(_hist(apop[...], NB, 8, False),
               t_all[_T_AP:_T_AP + 8]) * (1.0 / L)
    e_fol = dot(_hist(fol[...], FOL_B, 16, False),
                t_all[_T_FOL:_T_FOL + 16]) * (1.0 / L)
    e_tp = dot(_hist(tpop[...], TPOP_B, 16, False),
               t_all[_T_TP:_T_TP + 16]) * (1.0 / L)

    x0 = jnp.concatenate([
        name_e[...], e_cl, can_e[...], e_ns, e_na, e_nb,
        pooled(s_a, ia, 0), pooled(s_u, iu, 1), pooled(s_t, it, 2),
        e_dur, pooled(s_al, ial, 3),
        e_ap, e_fol, e_tp, pooled(s_g, ig, 4),
    ], axis=1)

    d0 = 15 * EMB
    c1 = dot(x0, U[...])
    c2 = dot(c1, V[...])
    x = x0 * (c2 + b_all[0:1, 0:d0]) + x0
    h = jnp.maximum(dot(x, W1[...]) + b_all[0:1, d0:d0 + 512], 0.0)
    h = jnp.maximum(dot(h, W2[...]) + b_all[0:1, d0 + 512:d0 + 768], 0.0)
    h = dot(h, W3[...]) + b_all[0:1, d0 + 768:d0 + 896]
    nrm = lax.rsqrt(jnp.maximum(jnp.sum(h * h, axis=1, keepdims=True), 1e-12))
    out[...] = h * nrm


def kernel(name, collaborative, track_uri_can, n_songs_pl, num_artists_pl,
           num_albums_pl, artist_name_pl, track_uri_pl, track_name_pl,
           duration_ms_songs_pl, album_name_pl, artist_pop_pl,
           artists_followers_pl, track_pop_pl, artist_genres_pl, params):
    p = params
    i32 = jnp.int32
    f32 = jnp.float32

    ia = artist_name_pl.astype(i32)
    iu = track_uri_pl.astype(i32)
    it = track_name_pl.astype(i32)
    ial = album_name_pl.astype(i32)
    ig = artist_genres_pl.astype(i32)
    iname = name.astype(i32)
    ican = track_uri_can.astype(i32)

    base = np.repeat(np.arange(BPW, dtype=np.int32), L).reshape(PNC, PCH)
    one = np.concatenate([base + f * BPW for f in range(NPOOL)], axis=0)
    one = np.concatenate(
        [one, np.zeros((NDEST - NPOOL * PNC, PCH), np.int32)], axis=0)
    dest = jnp.asarray(
        np.stack([one + s * (NPOOL * BPW) for s in range(16)], axis=0))
    zeros = jnp.zeros((NPOOL * BPW, EMB), f32)

    sc = _sc_kernel()
    nprow = BS * L // PCH

    sc_outs = []
    for h in range(NSPLIT):
        sl = slice(h * BS, (h + 1) * BS)
        sc_outs.append(sc(
            ia[sl].reshape(nprow, PCH), iu[sl].reshape(nprow, PCH),
            it[sl].reshape(nprow, PCH), ial[sl].reshape(nprow, PCH),
            ig[sl].reshape(nprow, PCH),
            iname[sl].reshape(BS // SCH, SCH), ican[sl].reshape(BS // SCH, SCH),
            dest, zeros,
            p['artist_name_emb'], p['track_uri_pl_emb'], p['track_name_emb'],
            p['album_name_emb'], p['genres_emb'], p['name_emb'],
            p['track_uri_can_emb']))

    row0s = jnp.pad(jnp.stack([
        p['artist_name_emb'][0], p['track_uri_pl_emb'][0],
        p['track_name_emb'][0], p['album_name_emb'][0],
        p['genres_emb'][0]]), ((0, 3), (0, 0)))

    z2 = jnp.zeros((2, EMB), f32)
    z3 = jnp.zeros((3, EMB), f32)
    z4 = jnp.zeros((4, EMB), f32)
    z5 = jnp.zeros((5, EMB), f32)
    t_all = jnp.concatenate([
        p['collab_emb'], z4,
        p['n_songs_emb'], z2, p['n_artists_emb'], z2, p['n_albums_emb'], z2,
        p['duration_emb'], z3,
        p['artist_pop_emb'], z2,
        p['followers_emb'], z5, p['track_pop_emb'], z5,
    ], axis=0)

    d0 = 15 * EMB
    b_all = jnp.concatenate(
        [p['cross_b'], p['b1'], p['b2'], p['b3']]).reshape(1, d0 + 896)
    b_all = jnp.pad(b_all, ((0, 7), (0, 0)))
    U = p['cross_U']
    V = p['cross_V']

    def bspec(shape2):
        return pl.BlockSpec((BB, shape2), lambda i: (i, 0))

    def wspec(shape):
        return pl.BlockSpec(shape, lambda i: tuple(0 for _ in shape))

    in_specs = (
        [bspec(EMB)] * 7
        + [bspec(L)] * 5
        + [bspec(1)] * 4
        + [bspec(L)] * 4
        + [wspec((8, EMB)), wspec((_T_ROWS, EMB)),
           wspec((d0, 100)), wspec((100, d0)), wspec((8, d0 + 896)),
           wspec((d0, 512)), wspec((512, 256)), wspec((256, 128))]
    )

    outs = []
    for h in range(NSPLIT):
        sl = slice(h * BS, (h + 1) * BS)
        s_a, s_u, s_t, s_al, s_g, name_e, can_e = sc_outs[h]
        outs.append(pl.pallas_call(
            _tc_body,
            grid=(BS // BB,),
            in_specs=in_specs,
            out_specs=pl.BlockSpec((BB, EMB), lambda i: (i, 0)),
            out_shape=jax.ShapeDtypeStruct((BS, EMB), f32),
        )(
            name_e, can_e, s_a, s_u, s_t, s_al, s_g,
            ia[sl], iu[sl], it[sl], ial[sl], ig[sl],
            collaborative.astype(i32)[sl].reshape(BS, 1),
            n_songs_pl.astype(f32)[sl].reshape(BS, 1),
            num_artists_pl.astype(f32)[sl].reshape(BS, 1),
            num_albums_pl.astype(f32)[sl].reshape(BS, 1),
            duration_ms_songs_pl.astype(f32)[sl],
            artist_pop_pl.astype(f32)[sl],
            artists_followers_pl.astype(f32)[sl],
            track_pop_pl.astype(f32)[sl],
            row0s, t_all, U, V, b_all,
            p['W1'], p['W2'], p['W3'],
        ))
    return jnp.concatenate(outs, axis=0)
